# baseline (speedup 1.0000x reference)
import functools

import jax
import jax.numpy as jnp
from jax import lax
from jax.experimental import pallas as pl
from jax.experimental.pallas import tpu as pltpu
from jax.experimental.pallas import tpu_sc as plsc

D_MODEL = 1024
NUM_WORKERS = 32
CHUNK = 32


NB = 3


def _gather_body(idx_hbm, table_hbm, out_hbm, idx_v, *rest):
    bufs = rest[:NB]
    gsems = rest[NB:2 * NB]
    wsems = rest[2 * NB:3 * NB]
    batch, seq_len = idx_hbm.shape
    w_per_b = NUM_WORKERS // batch
    b_per_w = seq_len // w_per_b
    nchunks = b_per_w // CHUNK
    nfull = nchunks // NB
    nrem = nchunks - nfull * NB
    wid = lax.axis_index("s") * 2 + lax.axis_index("c")
    b = wid // w_per_b
    row0 = (wid % w_per_b) * b_per_w
    pltpu.sync_copy(idx_hbm.at[b, pl.ds(row0, b_per_w)], idx_v)

    def g_copy(off, buf, sem):
        return pltpu.make_async_copy(
            table_hbm.at[idx_v.at[pl.ds(off, CHUNK)]], buf, sem)

    def w_copy(off, buf, sem):
        return pltpu.make_async_copy(
            buf.reshape(CHUNK, D_MODEL),
            out_hbm.at[b, pl.ds(row0 + off, CHUNK)], sem)

    def chunk_step(c, j, first=False):
        g_copy(c * CHUNK, bufs[j], gsems[j]).wait()
        w_copy(c * CHUNK, bufs[j], wsems[j]).start()
        if not first:
            w_copy((c - 1) * CHUNK, bufs[(j + 2) % NB],
                   wsems[(j + 2) % NB]).wait()
        g_copy((c + 2) * CHUNK, bufs[(j + 2) % NB], gsems[(j + 2) % NB]).start()

    g_copy(0, bufs[0], gsems[0]).start()
    g_copy(CHUNK, bufs[1], gsems[1]).start()
    chunk_step(0, 0, first=True)
    chunk_step(1, 1)
    chunk_step(2, 2)

    def tri(i, carry):
        c0 = i * NB
        for j in range(NB):
            c = c0 + j
            g_copy(c * CHUNK, bufs[j], gsems[j]).wait()
            w_copy(c * CHUNK, bufs[j], wsems[j]).start()
            w_copy((c - 1) * CHUNK, bufs[(j + 2) % NB],
                   wsems[(j + 2) % NB]).wait()
            g_copy((c + 2) * CHUNK, bufs[(j + 2) % NB],
                   gsems[(j + 2) % NB]).start()
        return carry

    lax.fori_loop(1, nchunks // NB, tri, 0)
    for c in (nchunks - 2, nchunks - 1):
        j = c % NB
        g_copy(c * CHUNK, bufs[j], gsems[j]).wait()
        w_copy(c * CHUNK, bufs[j], wsems[j]).start()
        w_copy((c - 1) * CHUNK, bufs[(j + 2) % NB],
               wsems[(j + 2) % NB]).wait()
    j = (nchunks - 1) % NB
    w_copy((nchunks - 1) * CHUNK, bufs[j], wsems[j]).wait()


def kernel(x, pe_table):
    batch, seq_len = x.shape
    mesh = plsc.VectorSubcoreMesh(core_axis_name="c", subcore_axis_name="s")
    gather = functools.partial(
        pl.kernel,
        mesh=mesh,
        out_type=jax.ShapeDtypeStruct((batch, seq_len, D_MODEL), jnp.float32),
        scratch_types=(
            [pltpu.VMEM((seq_len * batch // NUM_WORKERS,), jnp.int32)]
            + [pltpu.VMEM((CHUNK, 8, 128), jnp.float32) for _ in range(NB)]
            + [pltpu.SemaphoreType.DMA for _ in range(2 * NB)]
        ),
    )(_gather_body)
    t3 = pe_table.reshape(pe_table.shape[0], 8, 128)
    return gather(x, t3)


# speedup vs baseline: 1.2484x; 1.2484x over previous
import functools

import jax
import jax.numpy as jnp
from jax import lax
from jax.experimental import pallas as pl
from jax.experimental.pallas import tpu as pltpu
from jax.experimental.pallas import tpu_sc as plsc

D_MODEL = 1024
NUM_WORKERS = 32
CHUNK = 32
NB = 2


def _gather_body(idx_hbm, table_hbm, out_hbm, idx_v, *rest):
    bufs = rest[:NB]
    gsems = rest[NB:2 * NB]
    wsems = rest[2 * NB:3 * NB]
    batch, seq_len = idx_hbm.shape
    w_per_b = NUM_WORKERS // batch
    b_per_w = seq_len // w_per_b
    nchunks = b_per_w // CHUNK
    wid = lax.axis_index("s") * 2 + lax.axis_index("c")
    b = wid // w_per_b
    row0 = (wid % w_per_b) * b_per_w
    pltpu.sync_copy(idx_hbm.at[b, pl.ds(row0, b_per_w)], idx_v)

    def g_copy(off, buf, sem):
        return pltpu.make_async_copy(
            table_hbm.at[idx_v.at[pl.ds(off, CHUNK)]], buf, sem)

    def w_copy(off, buf, sem):
        return pltpu.make_async_copy(
            buf, out_hbm.at[b, pl.ds(row0 + off, CHUNK)], sem)

    buf0, buf1 = bufs[0], bufs[1]
    g0, g1 = gsems[0], gsems[1]
    w0, w1 = wsems[0], wsems[1]
    npairs = nchunks // 2
    g_copy(0, buf0, g0).start()

    def pair(i, carry):
        a = 2 * i * CHUNK

        @pl.when(i > 0)
        def _():
            w_copy(a - CHUNK, buf1, w1).wait()

        g_copy(a + CHUNK, buf1, g1).start()
        g_copy(a, buf0, g0).wait()
        w_copy(a, buf0, w0).start()

        @pl.when(i < npairs - 1)
        def _():
            w_copy(a, buf0, w0).wait()
            g_copy(a + 2 * CHUNK, buf0, g0).start()

        g_copy(a + CHUNK, buf1, g1).wait()
        w_copy(a + CHUNK, buf1, w1).start()
        return carry

    lax.fori_loop(0, npairs, pair, 0)
    last = (nchunks - 2) * CHUNK
    w_copy(last, buf0, w0).wait()
    w_copy(last + CHUNK, buf1, w1).wait()


def _sc_gather(x_sc, pe_table):
    batch, seq_len = x_sc.shape
    mesh = plsc.VectorSubcoreMesh(core_axis_name="c", subcore_axis_name="s")
    gather = functools.partial(
        pl.kernel,
        mesh=mesh,
        out_type=jax.ShapeDtypeStruct((batch, seq_len, D_MODEL), jnp.float32),
        scratch_types=(
            [pltpu.VMEM((seq_len * batch // NUM_WORKERS,), jnp.int32)]
            + [pltpu.VMEM((CHUNK, D_MODEL), jnp.float32) for _ in range(NB)]
            + [pltpu.SemaphoreType.DMA for _ in range(2 * NB)]
        ),
    )(_gather_body)
    return gather(x_sc, pe_table)


def kernel(x, pe_table):
    return _sc_gather(x, pe_table)
